# Initial kernel scaffold; baseline (speedup 1.0000x reference)
import jax
import jax.numpy as jnp
from jax.experimental import pallas as pl


def kernel(g, drop_feat1, drop_feat2, drop_g1, drop_g2, W1, b1, W2, b2, P1, pb1, prelu_a, P2, pb2):
    raise NotImplementedError("write your pallas kernel here")


# speedup vs baseline: 5.9926x; 5.9926x over previous
import functools

import jax
import jax.numpy as jnp
from jax import lax
from jax.experimental import pallas as pl
from jax.experimental.pallas import tpu as pltpu
from jax.experimental.pallas import tpu_sc as plsc

N = 10000
D = 128
H = 256
E_DROP = 160000

NC = 2
NS = 16

EB = 64
CHUNK = 10240
NBATCH = CHUNK // EB
E_PAD = NS * CHUNK
NBUF = 4
SUPER = 32
NSB = NBATCH // SUPER
NWIN = SUPER // NBUF
EBD = 128
NBATCH_D = CHUNK // EBD

N_PAD = 10112
RPT = N_PAD // NS

BLK = 1000
NB = N // BLK


def _sc_mesh():
  return plsc.VectorSubcoreMesh(core_axis_name="c", subcore_axis_name="s",
                                num_cores=NC, num_subcores=NS)


@functools.cache
def _deg_kernel():
  def body(dst12_h, zeros_h, ones_h, out_h, acc, dsts, onesv):
    c = lax.axis_index("c")
    t = lax.axis_index("s")
    pltpu.sync_copy(zeros_h, acc.at[pl.ds(t * RPT, RPT)])
    pltpu.sync_copy(ones_h, onesv)
    pltpu.sync_copy(dst12_h.at[c, pl.ds(t * NBATCH_D, NBATCH_D)], dsts)
    plsc.subcore_barrier()

    def step(i, carry):
      pltpu.sync_copy(onesv, acc.at[dsts.at[i]], add=True)
      return carry

    lax.fori_loop(0, NBATCH_D, step, 0)

    plsc.subcore_barrier()
    pltpu.sync_copy(acc.at[pl.ds(t * RPT, RPT)],
                    out_h.at[c, pl.ds(t * RPT, RPT)])

  return pl.kernel(
      body,
      out_type=jax.ShapeDtypeStruct((NC, N_PAD, 16), jnp.float32),
      mesh=_sc_mesh(),
      scratch_types=[
          pltpu.VMEM_SHARED((N_PAD, 16), jnp.float32),
          pltpu.VMEM((NBATCH_D, EBD), jnp.int32),
          pltpu.VMEM((EBD, 16), jnp.float32),
      ],
  )


@functools.cache
def _prop_kernel():
  CHUNKS = []
  off = 0
  while off + EB < RPT:
    CHUNKS.append(off)
    off += EB
  CHUNKS.append(RPT - EB)

  def body(hp_h, src_h, dst_h, out_h,
           acc, srcb0, srcb1, dstb0, dstb1, rows, gsems, ssems, isems):
    c = lax.axis_index("c")
    t = lax.axis_index("s")
    base = t * RPT
    srcbs = (srcb0, srcb1)
    dstbs = (dstb0, dstb1)

    def zrow(i, carry):
      for j in range(D // 16):
        rows[0, i, pl.ds(j * 16, 16)] = jnp.zeros((16,), jnp.float32)
      return carry

    lax.fori_loop(0, EB, zrow, 0)
    for off in CHUNKS:
      pltpu.sync_copy(rows.at[0], acc.at[pl.ds(base + off, EB)])

    def idx_start(s, sb):
      r0 = t * NBATCH + s * SUPER
      pltpu.async_copy(src_h.at[pl.ds(r0, SUPER)], srcbs[sb], isems.at[0])
      pltpu.async_copy(dst_h.at[pl.ds(r0, SUPER)], dstbs[sb], isems.at[1])

    def idx_wait(sb):
      pltpu.make_async_copy(src_h.at[pl.ds(0, SUPER)], srcbs[sb],
                            isems.at[0]).wait()
      pltpu.make_async_copy(dst_h.at[pl.ds(0, SUPER)], dstbs[sb],
                            isems.at[1]).wait()

    idx_start(0, 0)
    idx_wait(0)
    plsc.subcore_barrier()

    def mapk(sb, k):
      for j in range(EB // 16):
        v = srcbs[sb][k, pl.ds(j * 16, 16)]
        srcbs[sb][k, pl.ds(j * 16, 16)] = v + v + c

    def gather_start(sb, k, b):
      pltpu.async_copy(hp_h.at[srcbs[sb].at[k]], rows.at[b], gsems.at[b])

    def gather_wait(b):
      pltpu.make_async_copy(hp_h.at[srcbs[0].at[0]], rows.at[b],
                            gsems.at[b]).wait()

    def scatter_start(sb, k, b):
      pltpu.async_copy(rows.at[b], acc.at[dstbs[sb].at[k]], ssems.at[b],
                       add=True)

    def scatter_wait(b):
      pltpu.make_async_copy(rows.at[b], acc.at[dstbs[0].at[0]],
                            ssems.at[b]).wait()

    for s in range(NSB):
      sb = s % 2
      if s + 1 < NSB:
        idx_start(s + 1, (s + 1) % 2)
      for b in range(NBUF):
        mapk(sb, b)
        gather_start(sb, b, b)

      def window(g, carry):
        for b in range(NBUF):
          k = g * NBUF + b
          gather_wait(b)
          scatter_start(sb, k, b)
          scatter_wait(b)

          @pl.when(g < NWIN - 1)
          def _():
            k2 = (g + 1) * NBUF + b
            mapk(sb, k2)
            gather_start(sb, k2, b)

        return carry

      lax.fori_loop(0, NWIN, window, 0)
      if s + 1 < NSB:
        idx_wait((s + 1) % 2)

    plsc.subcore_barrier()

    for k, off in enumerate(CHUNKS):
      b = k % NBUF
      if k >= NBUF:
        pltpu.make_async_copy(rows.at[b], out_h.at[c, pl.ds(base, EB)],
                              gsems.at[b]).wait()
      pltpu.sync_copy(acc.at[pl.ds(base + off, EB)], rows.at[b])
      pltpu.async_copy(rows.at[b], out_h.at[c, pl.ds(base + off, EB)],
                       gsems.at[b])
    for b in range(NBUF):
      pltpu.make_async_copy(rows.at[b], out_h.at[c, pl.ds(base, EB)],
                            gsems.at[b]).wait()

  return pl.kernel(
      body,
      out_type=jax.ShapeDtypeStruct((NC, N_PAD, D), jnp.float32),
      mesh=_sc_mesh(),
      scratch_types=[
          pltpu.VMEM_SHARED((N_PAD, D), jnp.float32),
          pltpu.VMEM((SUPER, EB), jnp.int32),
          pltpu.VMEM((SUPER, EB), jnp.int32),
          pltpu.VMEM((SUPER, EB), jnp.int32),
          pltpu.VMEM((SUPER, EB), jnp.int32),
          pltpu.VMEM((NBUF, EB, D), jnp.float32),
          pltpu.SemaphoreType.DMA((NBUF,)),
          pltpu.SemaphoreType.DMA((NBUF,)),
          pltpu.SemaphoreType.DMA((2,)),
      ],
  )


def _dense1_body(x_ref, w_ref, b_ref, deg_ref, out_ref):
  s = lax.rsqrt(deg_ref[...] + 1.0)
  out_ref[...] = (jnp.dot(x_ref[...], w_ref[...],
                          preferred_element_type=jnp.float32)
                  + b_ref[...]) * s


def _dense1(x, w1, b1, deg):
  return pl.pallas_call(
      _dense1_body,
      grid=(NB,),
      in_specs=[
          pl.BlockSpec((BLK, D), lambda i: (i, 0)),
          pl.BlockSpec((D, H), lambda i: (0, 0)),
          pl.BlockSpec((1, H), lambda i: (0, 0)),
          pl.BlockSpec((BLK, 1), lambda i: (i, 0)),
      ],
      out_specs=pl.BlockSpec((BLK, H), lambda i: (i, 0)),
      out_shape=jax.ShapeDtypeStruct((N, H), jnp.float32),
  )(x, w1, b1, deg)


def _dense2_body(a0_ref, a1_ref, hp_ref, w_ref, b_ref, deg_ref, out_ref):
  s = lax.rsqrt(deg_ref[...] + 1.0)
  hp = hp_ref[...]
  h_lo = jnp.maximum((a0_ref[0] + hp[:, :D]) * s, 0.0)
  h_hi = jnp.maximum((a1_ref[0] + hp[:, D:]) * s, 0.0)
  acc = jnp.dot(h_lo, w_ref[:D, :], preferred_element_type=jnp.float32)
  acc += jnp.dot(h_hi, w_ref[D:, :], preferred_element_type=jnp.float32)
  out_ref[...] = (acc + b_ref[...]) * s


def _dense2(agg, hp, w2, b2, deg):
  return pl.pallas_call(
      _dense2_body,
      grid=(NB,),
      in_specs=[
          pl.BlockSpec((1, BLK, D), lambda i: (0, i, 0)),
          pl.BlockSpec((1, BLK, D), lambda i: (1, i, 0)),
          pl.BlockSpec((BLK, H), lambda i: (i, 0)),
          pl.BlockSpec((H, H), lambda i: (0, 0)),
          pl.BlockSpec((1, H), lambda i: (0, 0)),
          pl.BlockSpec((BLK, 1), lambda i: (i, 0)),
      ],
      out_specs=pl.BlockSpec((BLK, H), lambda i: (i, 0)),
      out_shape=jax.ShapeDtypeStruct((N, H), jnp.float32),
  )(agg, agg, hp, w2, b2, deg)


def _loss_body(a1l_ref, a1h_ref, hpa_ref, a2l_ref, a2h_ref, hpb_ref,
               deg1_ref, deg2_ref, p1_ref, pb1_ref, pa_ref, p2_ref, pb2_ref,
               out_ref):
  i = pl.program_id(0)
  s1 = lax.rsqrt(deg1_ref[...] + 1.0)
  s2 = lax.rsqrt(deg2_ref[...] + 1.0)
  hpa = hpa_ref[...]
  hpb = hpb_ref[...]
  e1_lo = (a1l_ref[0] + hpa[:, :D]) * s1
  e1_hi = (a1h_ref[0] + hpa[:, D:]) * s1
  e2_lo = (a2l_ref[0] + hpb[:, :D]) * s2
  e2_hi = (a2h_ref[0] + hpb[:, D:]) * s2
  a = pa_ref[0, 0]

  def predict(lo, hi):
    z = jnp.dot(lo, p1_ref[:D, :], preferred_element_type=jnp.float32)
    z += jnp.dot(hi, p1_ref[D:, :], preferred_element_type=jnp.float32)
    z += pb1_ref[...]
    z = jnp.where(z > 0, z, a * z)
    return jnp.dot(z, p2_ref[...],
                   preferred_element_type=jnp.float32) + pb2_ref[...]

  p1 = predict(e1_lo, e1_hi)
  p2 = predict(e2_lo, e2_hi)

  def cos(p, e_lo, e_hi):
    d = jnp.sum(p[:, :D] * e_lo + p[:, D:] * e_hi, axis=1)
    np_ = jnp.maximum(jnp.sqrt(jnp.sum(p * p, axis=1)), 1e-12)
    ne = jnp.maximum(jnp.sqrt(jnp.sum(e_lo * e_lo, axis=1)
                              + jnp.sum(e_hi * e_hi, axis=1)), 1e-12)
    return d / (np_ * ne)

  partial = jnp.sum(4.0 - 2.0 * cos(p1, e2_lo, e2_hi)
                    - 2.0 * cos(p2, e1_lo, e1_hi)) * (1.0 / N)

  @pl.when(i == 0)
  def _():
    out_ref[...] = jnp.zeros((1, 1), jnp.float32)

  out_ref[...] += jnp.full((1, 1), partial, jnp.float32)


def _loss(agg1, hp1, agg2, hp2, deg1, deg2, p1w, pb1, pa, p2w, pb2):
  return pl.pallas_call(
      _loss_body,
      grid=(NB,),
      in_specs=[
          pl.BlockSpec((1, BLK, D), lambda i: (0, i, 0)),
          pl.BlockSpec((1, BLK, D), lambda i: (1, i, 0)),
          pl.BlockSpec((BLK, H), lambda i: (i, 0)),
          pl.BlockSpec((1, BLK, D), lambda i: (0, i, 0)),
          pl.BlockSpec((1, BLK, D), lambda i: (1, i, 0)),
          pl.BlockSpec((BLK, H), lambda i: (i, 0)),
          pl.BlockSpec((BLK, 1), lambda i: (i, 0)),
          pl.BlockSpec((BLK, 1), lambda i: (i, 0)),
          pl.BlockSpec((H, H), lambda i: (0, 0)),
          pl.BlockSpec((1, H), lambda i: (0, 0)),
          pl.BlockSpec((1, 1), lambda i: (0, 0)),
          pl.BlockSpec((H, H), lambda i: (0, 0)),
          pl.BlockSpec((1, H), lambda i: (0, 0)),
      ],
      out_specs=pl.BlockSpec((1, 1), lambda i: (0, 0)),
      out_shape=jax.ShapeDtypeStruct((1, 1), jnp.float32),
  )(agg1, agg1, hp1, agg2, agg2, hp2, deg1, deg2, p1w, pb1, pa, p2w, pb2)


def _pad_edges(eidx):
  src = eidx[0].astype(jnp.int32)
  dst = eidx[1].astype(jnp.int32)
  pad = E_PAD - E_DROP
  src_p = jnp.concatenate([src, jnp.zeros((pad,), jnp.int32)])
  dst_p = jnp.concatenate([dst, jnp.full((pad,), N, jnp.int32)])
  return src_p.reshape(E_PAD // EB, EB), dst_p.reshape(E_PAD // EB, EB)


def kernel(g, drop_feat1, drop_feat2, drop_g1, drop_g2,
           W1, b1, W2, b2, P1, pb1, prelu_a, P2, pb2):
  src1, dst1 = _pad_edges(drop_g1)
  src2, dst2 = _pad_edges(drop_g2)

  deg1 = jnp.zeros((N_PAD,), jnp.float32).at[dst1.reshape(-1)].add(1.0)
  deg2 = jnp.zeros((N_PAD,), jnp.float32).at[dst2.reshape(-1)].add(1.0)
  deg1 = deg1[:N, None]
  deg2 = deg2[:N, None]

  b1r = b1.reshape(1, H)
  b2r = b2.reshape(1, H)
  pb1r = pb1.reshape(1, H)
  pb2r = pb2.reshape(1, H)
  par = jnp.reshape(prelu_a, (1, 1)).astype(jnp.float32)

  hp1 = _dense1(drop_feat1, W1, b1r, deg1)
  hp2 = _dense1(drop_feat2, W1, b1r, deg2)

  agg1 = _prop_kernel()(hp1.reshape(2 * N, D), src1, dst1)
  agg2 = _prop_kernel()(hp2.reshape(2 * N, D), src2, dst2)

  hq1 = _dense2(agg1, hp1, W2, b2r, deg1)
  hq2 = _dense2(agg2, hp2, W2, b2r, deg2)

  agg21 = _prop_kernel()(hq1.reshape(2 * N, D), src1, dst1)
  agg22 = _prop_kernel()(hq2.reshape(2 * N, D), src2, dst2)

  out = _loss(agg21, hq1, agg22, hq2, deg1, deg2, P1, pb1r, par, P2, pb2r)
  return out[0, 0]
